# baseline (speedup 1.0000x reference)
import functools
import math

import jax
import jax.numpy as jnp
from jax import lax
from jax.experimental import pallas as pl
from jax.experimental.pallas import tpu as pltpu
from jax.experimental.pallas import tpu_sc as plsc

_N = 10000
_E = 320000
_D = 128
_NBF = 12
_RH = 64
_L = 3
_C = float(0.5 ** 0.5)
_INV_NN = float(1.0 / math.sqrt(32.0))

_NW = 32
_CHUNK = 128
_CPW0 = 96
_CPW1 = 64
_G = 8
_EPAD = 16 * (_CPW0 + _CPW1) * _CHUNK
_NPAD = 10112
_RPS = _NPAD // 16
_RB = 400
_EB = 2560
_DUMP = _NPAD - 1


def _silu(v):
    return v / (1.0 + jnp.exp(-v))


def _wprime_body(esT_ref, attrT_ref, f1_ref, f2_ref, out_ref):
    es = esT_ref[...].T
    attr = attrT_ref[...].T
    hdn = jnp.dot(es, f1_ref[...], preferred_element_type=jnp.float32)
    w = jnp.dot(_silu(hdn), f2_ref[...], preferred_element_type=jnp.float32)
    out_ref[...] = w * attr


_wprime_call = pl.pallas_call(
    _wprime_body,
    grid=(_E // _EB,),
    in_specs=[
        pl.BlockSpec((_NBF, _EB), lambda e: (0, e)),
        pl.BlockSpec((1, _EB), lambda e: (0, e)),
        pl.BlockSpec((_NBF, _RH), lambda e: (0, 0)),
        pl.BlockSpec((_RH, _D), lambda e: (0, 0)),
    ],
    out_specs=pl.BlockSpec((_EB, _D), lambda e: (e, 0)),
    out_shape=jax.ShapeDtypeStruct((_EPAD, _D), jnp.float32),
)


def _dense0_body(x_ref, z_ref, wsc_ref, w1_ref, s_ref, t_ref):
    hz = x_ref[...] * z_ref[...]
    s_ref[...] = jnp.dot(hz, wsc_ref[...], preferred_element_type=jnp.float32)
    t_ref[...] = jnp.dot(hz, w1_ref[...], preferred_element_type=jnp.float32)


_dense0_call = pl.pallas_call(
    _dense0_body,
    grid=(_N // _RB,),
    in_specs=[
        pl.BlockSpec((_RB, _D), lambda i: (i, 0)),
        pl.BlockSpec((_RB, 1), lambda i: (i, 0)),
        pl.BlockSpec((_D, _D), lambda i: (0, 0)),
        pl.BlockSpec((_D, _D), lambda i: (0, 0)),
    ],
    out_specs=[pl.BlockSpec((_RB, _D), lambda i: (i, 0))] * 2,
    out_shape=[jax.ShapeDtypeStruct((_N, _D), jnp.float32)] * 2,
)


def _denseu_body(p0_ref, p1_ref, s_ref, z_ref, w2_ref, wsc_ref, w1_ref,
                 s_out, t_out):
    agg = (p0_ref[0] + p1_ref[0]) * _INV_NN
    conv = jnp.dot(agg * z_ref[...], w2_ref[...],
                   preferred_element_type=jnp.float32)
    h = _silu(_C * s_ref[...] + _C * conv)
    hz = h * z_ref[...]
    s_out[...] = jnp.dot(hz, wsc_ref[...], preferred_element_type=jnp.float32)
    t_out[...] = jnp.dot(hz, w1_ref[...], preferred_element_type=jnp.float32)


_denseu_call = pl.pallas_call(
    _denseu_body,
    grid=(_N // _RB,),
    in_specs=[
        pl.BlockSpec((1, _RB, _D), lambda i: (0, i, 0)),
        pl.BlockSpec((1, _RB, _D), lambda i: (1, i, 0)),
        pl.BlockSpec((_RB, _D), lambda i: (i, 0)),
        pl.BlockSpec((_RB, 1), lambda i: (i, 0)),
        pl.BlockSpec((_D, _D), lambda i: (0, 0)),
        pl.BlockSpec((_D, _D), lambda i: (0, 0)),
        pl.BlockSpec((_D, _D), lambda i: (0, 0)),
    ],
    out_specs=[pl.BlockSpec((_RB, _D), lambda i: (i, 0))] * 2,
    out_shape=[jax.ShapeDtypeStruct((_N, _D), jnp.float32)] * 2,
)


def _densef_body(p0_ref, p1_ref, s_ref, z_ref, w2_ref, h_out):
    agg = (p0_ref[0] + p1_ref[0]) * _INV_NN
    conv = jnp.dot(agg * z_ref[...], w2_ref[...],
                   preferred_element_type=jnp.float32)
    h_out[...] = _silu(_C * s_ref[...] + _C * conv)


_densef_call = pl.pallas_call(
    _densef_body,
    grid=(_N // _RB,),
    in_specs=[
        pl.BlockSpec((1, _RB, _D), lambda i: (0, i, 0)),
        pl.BlockSpec((1, _RB, _D), lambda i: (1, i, 0)),
        pl.BlockSpec((_RB, _D), lambda i: (i, 0)),
        pl.BlockSpec((_RB, 1), lambda i: (i, 0)),
        pl.BlockSpec((_D, _D), lambda i: (0, 0)),
    ],
    out_specs=pl.BlockSpec((_RB, _D), lambda i: (i, 0)),
    out_shape=jax.ShapeDtypeStruct((_N, _D), jnp.float32),
)


_sc_mesh = plsc.VectorSubcoreMesh(core_axis_name="c", subcore_axis_name="s")


@functools.partial(
    pl.kernel,
    out_type=jax.ShapeDtypeStruct((2, _NPAD, _D), jnp.float32),
    mesh=_sc_mesh,
    scratch_types=[
        pltpu.VMEM((_G, _CHUNK), jnp.int32),
        pltpu.VMEM((_G, _CHUNK), jnp.int32),
        pltpu.VMEM((_CHUNK, _D), jnp.float32),
        pltpu.VMEM((_CHUNK, _D), jnp.float32),
        pltpu.VMEM((_CHUNK // 2, _D), jnp.float32),
        pltpu.VMEM_SHARED((_NPAD, _D), jnp.float32),
        pltpu.SemaphoreType.DMA,
        pltpu.SemaphoreType.DMA,
        pltpu.SemaphoreType.DMA,
    ],
)
def _sc_call(t_hbm, w_hbm, src_hbm, dst_hbm, zeros_hbm, out_hbm,
                src_v, dst_v, rows0_v, rows1_v, w_v, agg_sh,
                sem_g0, sem_g1, sem_w):
    cid = lax.axis_index("c")
    sid = lax.axis_index("s")
    pltpu.sync_copy(zeros_hbm.at[pl.ds(sid * _RPS, _RPS)],
                    agg_sh.at[pl.ds(sid * _RPS, _RPS)])
    base = jnp.where(cid == 0, sid * _CPW0, 16 * _CPW0 + sid * _CPW1)
    ngroups = jnp.where(cid == 0, _CPW0 // _G, _CPW1 // _G)
    plsc.subcore_barrier()

    rows = (rows0_v, rows1_v)
    sems = (sem_g0, sem_g1)

    def group_body(g, carry):
        pltpu.sync_copy(src_hbm.at[pl.ds(base + g * _G, _G)], src_v)
        pltpu.sync_copy(dst_hbm.at[pl.ds(base + g * _G, _G)], dst_v)
        desc = [None, None]
        desc[0] = pltpu.async_copy(t_hbm.at[src_v.at[0]], rows[0], sems[0])
        for jj in range(_G):
            b = jj % 2
            if jj + 1 < _G:
                desc[1 - b] = pltpu.async_copy(
                    t_hbm.at[src_v.at[jj + 1]], rows[1 - b], sems[1 - b])
            row0 = (base + g * _G + jj) * _CHUNK
            wld = pltpu.async_copy(
                w_hbm.at[pl.ds(row0, _CHUNK // 2)], w_v, sem_w)
            desc[b].wait()
            rv = rows[b]
            for half in range(2):
                off = half * (_CHUNK // 2)
                wld.wait()

                @plsc.parallel_loop(0, _CHUNK // 2, step=1, unroll=2)
                def mbody(r, rv=rv, off=off):
                    for cb in range(_D // 16):
                        sl = pl.ds(cb * 16, 16)
                        rv[off + r, sl] = rv[off + r, sl] * w_v[r, sl]

                if half == 0:
                    wld = pltpu.async_copy(
                        w_hbm.at[pl.ds(row0 + _CHUNK // 2, _CHUNK // 2)],
                        w_v, sem_w)

            pltpu.sync_copy(rv, agg_sh.at[dst_v.at[jj]], add=True)
        return carry

    lax.fori_loop(0, ngroups, group_body, 0)
    plsc.subcore_barrier()
    pltpu.sync_copy(agg_sh.at[pl.ds(sid * _RPS, _RPS)],
                    out_hbm.at[cid, pl.ds(sid * _RPS, _RPS)])


def kernel(x, z, edge_src, edge_dst, edge_attr, edge_scalars, Wsc, W1, F1, F2, W2):
    pad = _EPAD - _E
    src2 = jnp.pad(edge_src, (0, pad)).reshape(_EPAD // _CHUNK, _CHUNK)
    dst2 = jnp.pad(edge_dst, (0, pad),
                   constant_values=_DUMP).reshape(_EPAD // _CHUNK, _CHUNK)
    zeros = jnp.zeros((_NPAD, _D), jnp.float32)
    esT = edge_scalars.T
    attrT = edge_attr.reshape(1, _E)
    wps = [_wprime_call(esT, attrT, F1[l], F2[l]) for l in range(_L)]
    s, t = _dense0_call(x, z, Wsc[0], W1[0])
    h = None
    for l in range(_L):
        aggp = _sc_call(t, wps[l], src2, dst2, zeros)
        if l + 1 < _L:
            s, t = _denseu_call(aggp, aggp, s, z, W2[l], Wsc[l + 1], W1[l + 1])
        else:
            h = _densef_call(aggp, aggp, s, z, W2[l])
    return h


# speedup vs baseline: 1.0763x; 1.0763x over previous
import functools
import math

import jax
import jax.numpy as jnp
from jax import lax
from jax.experimental import pallas as pl
from jax.experimental.pallas import tpu as pltpu
from jax.experimental.pallas import tpu_sc as plsc

_N = 10000
_E = 320000
_D = 128
_NBF = 12
_RH = 64
_L = 3
_C = float(0.5 ** 0.5)
_INV_NN = float(1.0 / math.sqrt(32.0))

_NW = 32
_CHUNK = 128
_CPW0 = 96
_CPW1 = 64
_G = 8
_EPAD = 16 * (_CPW0 + _CPW1) * _CHUNK
_NPAD = 10112
_RPS = _NPAD // 16
_RB = 400
_EB = 2560
_DUMP = _NPAD - 1


def _silu(v):
    return v / (1.0 + jnp.exp(-v))


def _wprime_body(esT_ref, attrT_ref, f1_ref, f2_ref, out_ref):
    es = esT_ref[...].T
    attr = attrT_ref[...].T
    for l in range(_L):
        hdn = jnp.dot(es, f1_ref[l], preferred_element_type=jnp.float32)
        w = jnp.dot(_silu(hdn), f2_ref[l], preferred_element_type=jnp.float32)
        out_ref[l] = w * attr


_wprime_call = pl.pallas_call(
    _wprime_body,
    grid=(_E // _EB,),
    in_specs=[
        pl.BlockSpec((_NBF, _EB), lambda e: (0, e)),
        pl.BlockSpec((1, _EB), lambda e: (0, e)),
        pl.BlockSpec((_L, _NBF, _RH), lambda e: (0, 0, 0)),
        pl.BlockSpec((_L, _RH, _D), lambda e: (0, 0, 0)),
    ],
    out_specs=pl.BlockSpec((_L, _EB, _D), lambda e: (0, e, 0)),
    out_shape=jax.ShapeDtypeStruct((_L, _EPAD, _D), jnp.float32),
)


def _dense0_body(x_ref, z_ref, wsc_ref, w1_ref, s_ref, t_ref):
    hz = x_ref[...] * z_ref[...]
    s_ref[...] = jnp.dot(hz, wsc_ref[...], preferred_element_type=jnp.float32)
    t_ref[...] = jnp.dot(hz, w1_ref[...], preferred_element_type=jnp.float32)


_dense0_call = pl.pallas_call(
    _dense0_body,
    grid=(_N // _RB,),
    in_specs=[
        pl.BlockSpec((_RB, _D), lambda i: (i, 0)),
        pl.BlockSpec((_RB, 1), lambda i: (i, 0)),
        pl.BlockSpec((_D, _D), lambda i: (0, 0)),
        pl.BlockSpec((_D, _D), lambda i: (0, 0)),
    ],
    out_specs=[pl.BlockSpec((_RB, _D), lambda i: (i, 0))] * 2,
    out_shape=[jax.ShapeDtypeStruct((_N, _D), jnp.float32)] * 2,
)


def _denseu_body(p0_ref, p1_ref, s_ref, z_ref, w2_ref, wsc_ref, w1_ref,
                 s_out, t_out):
    agg = (p0_ref[0] + p1_ref[0]) * _INV_NN
    conv = jnp.dot(agg * z_ref[...], w2_ref[...],
                   preferred_element_type=jnp.float32)
    h = _silu(_C * s_ref[...] + _C * conv)
    hz = h * z_ref[...]
    s_out[...] = jnp.dot(hz, wsc_ref[...], preferred_element_type=jnp.float32)
    t_out[...] = jnp.dot(hz, w1_ref[...], preferred_element_type=jnp.float32)


_denseu_call = pl.pallas_call(
    _denseu_body,
    grid=(_N // _RB,),
    in_specs=[
        pl.BlockSpec((1, _RB, _D), lambda i: (0, i, 0)),
        pl.BlockSpec((1, _RB, _D), lambda i: (1, i, 0)),
        pl.BlockSpec((_RB, _D), lambda i: (i, 0)),
        pl.BlockSpec((_RB, 1), lambda i: (i, 0)),
        pl.BlockSpec((_D, _D), lambda i: (0, 0)),
        pl.BlockSpec((_D, _D), lambda i: (0, 0)),
        pl.BlockSpec((_D, _D), lambda i: (0, 0)),
    ],
    out_specs=[pl.BlockSpec((_RB, _D), lambda i: (i, 0))] * 2,
    out_shape=[jax.ShapeDtypeStruct((_N, _D), jnp.float32)] * 2,
)


def _densef_body(p0_ref, p1_ref, s_ref, z_ref, w2_ref, h_out):
    agg = (p0_ref[0] + p1_ref[0]) * _INV_NN
    conv = jnp.dot(agg * z_ref[...], w2_ref[...],
                   preferred_element_type=jnp.float32)
    h_out[...] = _silu(_C * s_ref[...] + _C * conv)


_densef_call = pl.pallas_call(
    _densef_body,
    grid=(_N // _RB,),
    in_specs=[
        pl.BlockSpec((1, _RB, _D), lambda i: (0, i, 0)),
        pl.BlockSpec((1, _RB, _D), lambda i: (1, i, 0)),
        pl.BlockSpec((_RB, _D), lambda i: (i, 0)),
        pl.BlockSpec((_RB, 1), lambda i: (i, 0)),
        pl.BlockSpec((_D, _D), lambda i: (0, 0)),
    ],
    out_specs=pl.BlockSpec((_RB, _D), lambda i: (i, 0)),
    out_shape=jax.ShapeDtypeStruct((_N, _D), jnp.float32),
)


_sc_mesh = plsc.VectorSubcoreMesh(core_axis_name="c", subcore_axis_name="s")


def _make_sc_call(l):
    @functools.partial(
        pl.kernel,
        out_type=jax.ShapeDtypeStruct((2, _NPAD, _D), jnp.float32),
        mesh=_sc_mesh,
        scratch_types=[
            pltpu.VMEM((_G, _CHUNK), jnp.int32),
            pltpu.VMEM((_G, _CHUNK), jnp.int32),
            pltpu.VMEM((_CHUNK, _D), jnp.float32),
            pltpu.VMEM((_CHUNK, _D), jnp.float32),
            pltpu.VMEM((_CHUNK // 2, _D), jnp.float32),
            pltpu.VMEM_SHARED((_NPAD, _D), jnp.float32),
            pltpu.SemaphoreType.DMA,
            pltpu.SemaphoreType.DMA,
            pltpu.SemaphoreType.DMA,
        ],
    )
    def _sc_scatter(t_hbm, w_hbm, src_hbm, dst_hbm, zeros_hbm, out_hbm,
                    src_v, dst_v, rows0_v, rows1_v, w_v, agg_sh,
                    sem_g0, sem_g1, sem_w):
        cid = lax.axis_index("c")
        sid = lax.axis_index("s")
        pltpu.sync_copy(zeros_hbm.at[pl.ds(sid * _RPS, _RPS)],
                        agg_sh.at[pl.ds(sid * _RPS, _RPS)])
        base = jnp.where(cid == 0, sid * _CPW0, 16 * _CPW0 + sid * _CPW1)
        ngroups = jnp.where(cid == 0, _CPW0 // _G, _CPW1 // _G)
        plsc.subcore_barrier()

        rows = (rows0_v, rows1_v)
        sems = (sem_g0, sem_g1)

        def group_body(g, carry):
            pltpu.sync_copy(src_hbm.at[pl.ds(base + g * _G, _G)], src_v)
            pltpu.sync_copy(dst_hbm.at[pl.ds(base + g * _G, _G)], dst_v)
            desc = [None, None]
            desc[0] = pltpu.async_copy(t_hbm.at[src_v.at[0]], rows[0], sems[0])
            for jj in range(_G):
                b = jj % 2
                if jj + 1 < _G:
                    desc[1 - b] = pltpu.async_copy(
                        t_hbm.at[src_v.at[jj + 1]], rows[1 - b], sems[1 - b])
                row0 = (base + g * _G + jj) * _CHUNK
                wld = pltpu.async_copy(
                    w_hbm.at[l, pl.ds(row0, _CHUNK // 2)], w_v, sem_w)
                desc[b].wait()
                rv = rows[b]
                for half in range(2):
                    off = half * (_CHUNK // 2)
                    wld.wait()

                    @plsc.parallel_loop(0, _CHUNK // 2, step=1, unroll=2)
                    def mbody(r, rv=rv, off=off):
                        for cb in range(_D // 16):
                            sl = pl.ds(cb * 16, 16)
                            rv[off + r, sl] = rv[off + r, sl] * w_v[r, sl]

                    if half == 0:
                        wld = pltpu.async_copy(
                            w_hbm.at[l, pl.ds(row0 + _CHUNK // 2, _CHUNK // 2)],
                            w_v, sem_w)

                pltpu.sync_copy(rv, agg_sh.at[dst_v.at[jj]], add=True)
            return carry

        lax.fori_loop(0, ngroups, group_body, 0)
        plsc.subcore_barrier()
        pltpu.sync_copy(agg_sh.at[pl.ds(sid * _RPS, _RPS)],
                        out_hbm.at[cid, pl.ds(sid * _RPS, _RPS)])

    return _sc_scatter


_sc_calls = [_make_sc_call(l) for l in range(_L)]


def kernel(x, z, edge_src, edge_dst, edge_attr, edge_scalars, Wsc, W1, F1, F2, W2):
    pad = _EPAD - _E
    src2 = jnp.pad(edge_src, (0, pad)).reshape(_EPAD // _CHUNK, _CHUNK)
    dst2 = jnp.pad(edge_dst, (0, pad),
                   constant_values=_DUMP).reshape(_EPAD // _CHUNK, _CHUNK)
    zeros = jnp.zeros((_NPAD, _D), jnp.float32)
    wp = _wprime_call(edge_scalars.T, edge_attr.reshape(1, _E), F1, F2)
    s, t = _dense0_call(x, z, Wsc[0], W1[0])
    h = None
    for l in range(_L):
        aggp = _sc_calls[l](t, wp, src2, dst2, zeros)
        if l + 1 < _L:
            s, t = _denseu_call(aggp, aggp, s, z, W2[l], Wsc[l + 1], W1[l + 1])
        else:
            h = _densef_call(aggp, aggp, s, z, W2[l])
    return h
